# Initial kernel scaffold; baseline (speedup 1.0000x reference)
import jax
import jax.numpy as jnp
from jax.experimental import pallas as pl


def kernel(x, state, Wq, Wk, Wv, gamma, Wgate, Wdown, ln1_w, ln1_b, ln2_w, ln2_b):
    raise NotImplementedError("write your pallas kernel here")


# speedup vs baseline: 239.4177x; 239.4177x over previous
import functools

import jax
import jax.numpy as jnp
from jax.experimental import pallas as pl
from jax.experimental.pallas import tpu as pltpu

EPS = 1e-5


def _ln_rows(xr, w, b):
    m = jnp.mean(xr, axis=-1, keepdims=True)
    xc = xr - m
    v = jnp.mean(xc * xc, axis=-1, keepdims=True)
    return xc * jax.lax.rsqrt(v + EPS) * w + b


def _qkv_body(x_ref, wq_ref, wk_ref, wv_ref, lw_ref, lb_ref, q_ref, k_ref, v_ref):
    xn = _ln_rows(x_ref[...], lw_ref[...], lb_ref[...])
    q_ref[...] = jnp.dot(xn, wq_ref[...], preferred_element_type=jnp.float32)
    k_ref[...] = jnp.dot(xn, wk_ref[...], preferred_element_type=jnp.float32)
    v_ref[...] = jnp.dot(xn, wv_ref[...], preferred_element_type=jnp.float32)


def _attn_body(nc, dh, x_ref, q_ref, k_ref, v_ref, g_ref, s0_ref, gf_ref,
               y_ref, fs_ref, st_ref):
    c = pl.program_id(1)

    @pl.when(c == 0)
    def _():
        st_ref[...] = s0_ref[...]

    cs = q_ref.shape[0]
    row = jax.lax.broadcasted_iota(jnp.int32, (cs, cs), 0)
    col = jax.lax.broadcasted_iota(jnp.int32, (cs, cs), 1)
    causal = row >= col
    tmul = (c * cs + 1 + jax.lax.broadcasted_iota(jnp.int32, (cs, 1), 0)
            ).astype(jnp.float32)

    outs = []
    for j in range(2):
        qj = q_ref[:, j * dh:(j + 1) * dh]
        kj = k_ref[:, j * dh:(j + 1) * dh]
        vj = v_ref[:, j * dh:(j + 1) * dh]
        s = jax.lax.dot_general(qj, vj, (((1,), (1,)), ((), ())),
                                preferred_element_type=jnp.float32)
        sm = jnp.where(causal, s, 0.0)
        intra = jnp.dot(sm, kj, preferred_element_type=jnp.float32)
        inter = jax.lax.dot_general(qj, st_ref[j], (((1,), (1,)), ((), ())),
                                    preferred_element_type=jnp.float32)
        qs = jnp.sum(qj, axis=1, keepdims=True)
        og = (tmul * qs) * g_ref[j]
        outs.append(intra + inter + og)
        st_ref[j] = st_ref[j] + jax.lax.dot_general(
            kj, vj, (((0,), (0,)), ((), ())),
            preferred_element_type=jnp.float32)

    y_ref[...] = x_ref[...] + jnp.concatenate(outs, axis=1)

    @pl.when(c == nc - 1)
    def _():
        fs_ref[...] = st_ref[...] + gf_ref[...]


def _mlp_up_body(y_ref, w1_ref, w2_ref, lw_ref, lb_ref, a_ref):
    x2 = _ln_rows(y_ref[...], lw_ref[...], lb_ref[...])
    gate = jnp.dot(x2, w1_ref[...], preferred_element_type=jnp.float32)
    up = jnp.dot(x2, w2_ref[...], preferred_element_type=jnp.float32)
    a_ref[...] = jax.nn.silu(up) * jax.nn.sigmoid(gate)


def _down_body(a_ref, wd_ref, y_ref, o_ref):
    o_ref[...] = y_ref[...] + jnp.dot(a_ref[...], wd_ref[...],
                                      preferred_element_type=jnp.float32)


def kernel(x, state, Wq, Wk, Wv, gamma, Wgate, Wdown, ln1_w, ln1_b,
           ln2_w, ln2_b):
    t, d = x.shape
    h, dh, _ = state.shape
    f32 = jnp.float32

    bt = min(256, t)
    bn = min(512, d)
    cs = min(256, t)
    nbm, nbn, nc, hp = t // bt, d // bn, t // cs, h // 2

    ln1w = ln1_w.reshape(1, d)
    ln1b = ln1_b.reshape(1, d)
    ln2w = ln2_w.reshape(1, d)
    ln2b = ln2_b.reshape(1, d)
    gamma3 = gamma.reshape(h, 1, dh)
    gfin = jnp.broadcast_to((t * gamma)[:, :, None], (h, dh, dh))

    cp = pltpu.CompilerParams(
        dimension_semantics=("parallel", "arbitrary"),
        vmem_limit_bytes=100 * 1024 * 1024,
    )

    q, k, v = pl.pallas_call(
        _qkv_body,
        grid=(nbn, nbm),
        in_specs=[
            pl.BlockSpec((bt, d), lambda n, m: (m, 0)),
            pl.BlockSpec((d, bn), lambda n, m: (0, n)),
            pl.BlockSpec((d, bn), lambda n, m: (0, n)),
            pl.BlockSpec((d, bn), lambda n, m: (0, n)),
            pl.BlockSpec((1, d), lambda n, m: (0, 0)),
            pl.BlockSpec((1, d), lambda n, m: (0, 0)),
        ],
        out_specs=[
            pl.BlockSpec((bt, bn), lambda n, m: (m, n)),
            pl.BlockSpec((bt, bn), lambda n, m: (m, n)),
            pl.BlockSpec((bt, bn), lambda n, m: (m, n)),
        ],
        out_shape=[jax.ShapeDtypeStruct((t, d), f32)] * 3,
        compiler_params=cp,
    )(x, Wq, Wk, Wv, ln1w, ln1b)

    y, fs = pl.pallas_call(
        functools.partial(_attn_body, nc, dh),
        grid=(hp, nc),
        in_specs=[
            pl.BlockSpec((cs, 2 * dh), lambda p, c: (c, p)),
            pl.BlockSpec((cs, 2 * dh), lambda p, c: (c, p)),
            pl.BlockSpec((cs, 2 * dh), lambda p, c: (c, p)),
            pl.BlockSpec((cs, 2 * dh), lambda p, c: (c, p)),
            pl.BlockSpec((2, 1, dh), lambda p, c: (p, 0, 0)),
            pl.BlockSpec((2, dh, dh), lambda p, c: (p, 0, 0)),
            pl.BlockSpec((2, dh, dh), lambda p, c: (p, 0, 0)),
        ],
        out_specs=[
            pl.BlockSpec((cs, 2 * dh), lambda p, c: (c, p)),
            pl.BlockSpec((2, dh, dh), lambda p, c: (p, 0, 0)),
        ],
        out_shape=[
            jax.ShapeDtypeStruct((t, d), f32),
            jax.ShapeDtypeStruct((h, dh, dh), f32),
        ],
        scratch_shapes=[pltpu.VMEM((2, dh, dh), f32)],
        compiler_params=cp,
    )(x, q, k, v, gamma3, state, gfin)

    a = pl.pallas_call(
        _mlp_up_body,
        grid=(nbn, nbm),
        in_specs=[
            pl.BlockSpec((bt, d), lambda n, m: (m, 0)),
            pl.BlockSpec((d, bn), lambda n, m: (0, n)),
            pl.BlockSpec((d, bn), lambda n, m: (0, nbn + n)),
            pl.BlockSpec((1, d), lambda n, m: (0, 0)),
            pl.BlockSpec((1, d), lambda n, m: (0, 0)),
        ],
        out_specs=pl.BlockSpec((bt, bn), lambda n, m: (m, n)),
        out_shape=jax.ShapeDtypeStruct((t, d), f32),
        compiler_params=cp,
    )(y, Wgate, Wgate, ln2w, ln2b)

    out = pl.pallas_call(
        _down_body,
        grid=(nbn, nbm),
        in_specs=[
            pl.BlockSpec((bt, d), lambda n, m: (m, 0)),
            pl.BlockSpec((d, bn), lambda n, m: (0, n)),
            pl.BlockSpec((bt, bn), lambda n, m: (m, n)),
        ],
        out_specs=pl.BlockSpec((bt, bn), lambda n, m: (m, n)),
        out_shape=jax.ShapeDtypeStruct((t, d), f32),
        compiler_params=cp,
    )(a, Wdown, y)

    return out, fs
